# baseline (speedup 1.0000x reference)
import numpy as np
import jax
import jax.numpy as jnp
from jax.experimental import pallas as pl
from jax.experimental.pallas import tpu as pltpu
from jax.sharding import Mesh, PartitionSpec as P

_W = 0.1
_NBINS = 64
_EPS = 1e-08
_LOG2E = 1.4426950408889634


def _kl_body(pred_ref, targ_ref, out_ref, ys_p, ys_t, hist_p, hist_t):
    t = targ_ref[0]
    p = pred_ref[0]

    vmin = jnp.min(t)
    vmax = jnp.max(t)
    denom = vmax - vmin + _EPS
    w = 1.0 / _NBINS
    s = jnp.sqrt(jnp.float32(_LOG2E)) / (jnp.sqrt(jnp.float32(2.0)) * w * denom)

    ys_t[...] = t * s
    ys_p[...] = p * s

    def bin_body(b, _):
        c = (b.astype(jnp.float32) + 0.5) * w
        mbs = (vmin + c * denom) * s
        yt = ys_t[...]
        yp = ys_p[...]
        et = jnp.exp2((yt - mbs) * (mbs - yt))
        ep = jnp.exp2((yp - mbs) * (mbs - yp))
        hist_t[pl.ds(b, 1), :] = jnp.sum(et, axis=0, keepdims=True)
        hist_p[pl.ds(b, 1), :] = jnp.sum(ep, axis=0, keepdims=True)
        return 0

    jax.lax.fori_loop(0, _NBINS, bin_body, 0)

    ht = jnp.sum(hist_t[...], axis=1, keepdims=True)
    hp = jnp.sum(hist_p[...], axis=1, keepdims=True)
    tp = ht / (jnp.sum(ht) + _EPS)
    pp = hp / (jnp.sum(hp) + _EPS)
    kl = jnp.sum(tp * (jnp.log(tp + _EPS) - jnp.log(pp + _EPS)))
    out_ref[0] = jnp.full((8, 128), kl, dtype=jnp.float32)


def _kl_pallas(p3, t3):
    b, rows, lanes = p3.shape
    return pl.pallas_call(
        _kl_body,
        out_shape=jax.ShapeDtypeStruct((b, 8, 128), jnp.float32),
        grid=(b,),
        in_specs=[
            pl.BlockSpec((1, rows, lanes), lambda i: (i, 0, 0)),
            pl.BlockSpec((1, rows, lanes), lambda i: (i, 0, 0)),
        ],
        out_specs=pl.BlockSpec((1, 8, 128), lambda i: (i, 0, 0)),
        scratch_shapes=[
            pltpu.VMEM((rows, lanes), jnp.float32),
            pltpu.VMEM((rows, lanes), jnp.float32),
            pltpu.VMEM((_NBINS, 128), jnp.float32),
            pltpu.VMEM((_NBINS, 128), jnp.float32),
        ],
        compiler_params=pltpu.CompilerParams(
            dimension_semantics=("parallel",),
        ),
        name="kl_soft_hist",
    )(p3, t3)


def kernel(pred, target):
    B = pred.shape[0]
    n = pred.size // B
    lanes = 128
    rows = n // lanes
    p3 = pred.reshape(B, rows, lanes)
    t3 = target.reshape(B, rows, lanes)

    devs = jax.devices()
    nd = min(len(devs), B)
    while B % nd:
        nd -= 1
    if nd > 1:
        mesh = Mesh(np.array(devs[:nd]), ("b",))
        out = jax.shard_map(
            _kl_pallas, mesh=mesh, in_specs=(P("b"), P("b")), out_specs=P("b"),
            check_vma=False,
        )(p3, t3)
    else:
        out = _kl_pallas(p3, t3)

    return _W * jnp.mean(out[:, 0, 0])


# speedup vs baseline: 1.7399x; 1.7399x over previous
import jax
import jax.numpy as jnp
from jax.experimental import pallas as pl
from jax.experimental.pallas import tpu as pltpu

_W = 0.1
_NBINS = 64
_EPS = 1e-08
_LOG2E = 1.4426950408889634


def _kl_body(pred_ref, targ_ref, out_ref, hist_p, hist_t):
    t = targ_ref[0]
    p = pred_ref[0]
    rows = t.shape[0]
    ch = 64
    nch = rows // ch

    vmin = jnp.min(t)
    vmax = jnp.max(t)
    denom = vmax - vmin + _EPS
    w = 1.0 / _NBINS
    inv_denom = 1.0 / denom
    s2 = jnp.float32(_LOG2E / (2.0 * w * w)) * inv_denom * inv_denom
    ns2 = -s2
    step = denom * w

    def bin_body(b, _):
        m = vmin + (b.astype(jnp.float32) + 0.5) * step
        acc_t = jnp.zeros((ch, 128), jnp.float32)
        acc_p = jnp.zeros((ch, 128), jnp.float32)
        for i in range(nch):
            tc = targ_ref[0, i * ch:(i + 1) * ch, :]
            pc = pred_ref[0, i * ch:(i + 1) * ch, :]
            ut = tc - m
            acc_t = acc_t + jnp.exp2(ut * (ut * ns2))
            up = pc - m
            acc_p = acc_p + jnp.exp2(up * (up * ns2))
        hist_t[pl.ds(b, 1), :] = jnp.sum(acc_t, axis=0, keepdims=True)
        hist_p[pl.ds(b, 1), :] = jnp.sum(acc_p, axis=0, keepdims=True)
        return 0

    jax.lax.fori_loop(0, _NBINS, bin_body, 0)

    ht = jnp.sum(hist_t[...], axis=1, keepdims=True)
    hp = jnp.sum(hist_p[...], axis=1, keepdims=True)
    tp = ht / (jnp.sum(ht) + _EPS)
    pp = hp / (jnp.sum(hp) + _EPS)
    kl = jnp.sum(tp * (jnp.log(tp + _EPS) - jnp.log(pp + _EPS)))
    out_ref[0] = jnp.full((8, 128), kl, dtype=jnp.float32)


def _kl_pallas(p3, t3):
    b, rows, lanes = p3.shape
    return pl.pallas_call(
        _kl_body,
        out_shape=jax.ShapeDtypeStruct((b, 8, 128), jnp.float32),
        grid=(b,),
        in_specs=[
            pl.BlockSpec((1, rows, lanes), lambda i: (i, 0, 0)),
            pl.BlockSpec((1, rows, lanes), lambda i: (i, 0, 0)),
        ],
        out_specs=pl.BlockSpec((1, 8, 128), lambda i: (i, 0, 0)),
        scratch_shapes=[
            pltpu.VMEM((_NBINS, 128), jnp.float32),
            pltpu.VMEM((_NBINS, 128), jnp.float32),
        ],
        compiler_params=pltpu.CompilerParams(
            dimension_semantics=("parallel",),
        ),
        name="kl_soft_hist",
    )(p3, t3)


def kernel(pred, target):
    B = pred.shape[0]
    n = pred.size // B
    lanes = 128
    rows = n // lanes
    p3 = pred.reshape(B, rows, lanes)
    t3 = target.reshape(B, rows, lanes)

    out = _kl_pallas(p3, t3)

    return _W * jnp.mean(out[:, 0, 0])


# speedup vs baseline: 1.7770x; 1.0213x over previous
import jax
import jax.numpy as jnp
from jax.experimental import pallas as pl
from jax.experimental.pallas import tpu as pltpu

_W = 0.1
_NBINS = 64
_EPS = 1e-08
_LOG2E = 1.4426950408889634


def _kl_body(pred_ref, targ_ref, out_ref, hist_p, hist_t):
    t = targ_ref[0]
    p = pred_ref[0]
    rows = t.shape[0]
    ch = 64
    nch = rows // ch

    vmin = jnp.min(t)
    vmax = jnp.max(t)
    denom = vmax - vmin + _EPS
    w = 1.0 / _NBINS
    inv_denom = 1.0 / denom
    s2 = jnp.float32(_LOG2E / (2.0 * w * w)) * inv_denom * inv_denom
    ns2 = -s2
    step = denom * w

    def bin_body(b, _):
        m = vmin + (b.astype(jnp.float32) + 0.5) * step
        acc_t = jnp.zeros((ch, 128), jnp.float32)
        acc_p = jnp.zeros((ch, 128), jnp.float32)
        for i in range(nch):
            tc = targ_ref[0, i * ch:(i + 1) * ch, :]
            pc = pred_ref[0, i * ch:(i + 1) * ch, :]
            ut = tc - m
            acc_t = acc_t + jnp.exp2(ut * (ut * ns2))
            up = pc - m
            acc_p = acc_p + jnp.exp2(up * (up * ns2))
        hist_t[pl.ds(b, 1), :] = jnp.sum(acc_t, axis=0, keepdims=True)
        hist_p[pl.ds(b, 1), :] = jnp.sum(acc_p, axis=0, keepdims=True)
        return 0

    jax.lax.fori_loop(0, _NBINS, bin_body, 0, unroll=2)

    ht = jnp.sum(hist_t[...], axis=1, keepdims=True)
    hp = jnp.sum(hist_p[...], axis=1, keepdims=True)
    tp = ht / (jnp.sum(ht) + _EPS)
    pp = hp / (jnp.sum(hp) + _EPS)
    kl = jnp.sum(tp * (jnp.log(tp + _EPS) - jnp.log(pp + _EPS)))
    out_ref[0] = jnp.full((8, 128), kl, dtype=jnp.float32)


def _kl_pallas(p3, t3):
    b, rows, lanes = p3.shape
    return pl.pallas_call(
        _kl_body,
        out_shape=jax.ShapeDtypeStruct((b, 8, 128), jnp.float32),
        grid=(b,),
        in_specs=[
            pl.BlockSpec((1, rows, lanes), lambda i: (i, 0, 0)),
            pl.BlockSpec((1, rows, lanes), lambda i: (i, 0, 0)),
        ],
        out_specs=pl.BlockSpec((1, 8, 128), lambda i: (i, 0, 0)),
        scratch_shapes=[
            pltpu.VMEM((_NBINS, 128), jnp.float32),
            pltpu.VMEM((_NBINS, 128), jnp.float32),
        ],
        compiler_params=pltpu.CompilerParams(
            dimension_semantics=("parallel",),
        ),
        name="kl_soft_hist",
    )(p3, t3)


def kernel(pred, target):
    B = pred.shape[0]
    n = pred.size // B
    lanes = 128
    rows = n // lanes
    p3 = pred.reshape(B, rows, lanes)
    t3 = target.reshape(B, rows, lanes)

    out = _kl_pallas(p3, t3)

    return _W * jnp.mean(out[:, 0, 0])


# speedup vs baseline: 1.7964x; 1.0109x over previous
import jax
import jax.numpy as jnp
from jax.experimental import pallas as pl
from jax.experimental.pallas import tpu as pltpu

_W = 0.1
_NBINS = 64
_EPS = 1e-08
_LOG2E = 1.4426950408889634


def _kl_body(pred_ref, targ_ref, out_ref, hist_p, hist_t):
    t = targ_ref[0]
    p = pred_ref[0]
    rows = t.shape[0]
    ch = 64
    nch = rows // ch

    vmin = jnp.min(t)
    vmax = jnp.max(t)
    denom = vmax - vmin + _EPS
    w = 1.0 / _NBINS
    inv_denom = 1.0 / denom
    s2 = jnp.float32(_LOG2E / (2.0 * w * w)) * inv_denom * inv_denom
    ns2 = -s2
    step = denom * w

    def bin_body(b, _):
        m = vmin + (b.astype(jnp.float32) + 0.5) * step
        acc_t = jnp.zeros((ch, 128), jnp.float32)
        acc_p = jnp.zeros((ch, 128), jnp.float32)
        for i in range(nch):
            tc = targ_ref[0, i * ch:(i + 1) * ch, :]
            pc = pred_ref[0, i * ch:(i + 1) * ch, :]
            ut = tc - m
            acc_t = acc_t + jnp.exp2(ut * (ut * ns2))
            up = pc - m
            acc_p = acc_p + jnp.exp2(up * (up * ns2))
        hist_t[pl.ds(b, 1), :] = jnp.sum(acc_t, axis=0, keepdims=True)
        hist_p[pl.ds(b, 1), :] = jnp.sum(acc_p, axis=0, keepdims=True)
        return 0

    jax.lax.fori_loop(0, _NBINS, bin_body, 0, unroll=4)

    ht = jnp.sum(hist_t[...], axis=1, keepdims=True)
    hp = jnp.sum(hist_p[...], axis=1, keepdims=True)
    tp = ht / (jnp.sum(ht) + _EPS)
    pp = hp / (jnp.sum(hp) + _EPS)
    kl = jnp.sum(tp * (jnp.log(tp + _EPS) - jnp.log(pp + _EPS)))
    out_ref[0] = jnp.full((8, 128), kl, dtype=jnp.float32)


def _kl_pallas(p3, t3):
    b, rows, lanes = p3.shape
    return pl.pallas_call(
        _kl_body,
        out_shape=jax.ShapeDtypeStruct((b, 8, 128), jnp.float32),
        grid=(b,),
        in_specs=[
            pl.BlockSpec((1, rows, lanes), lambda i: (i, 0, 0)),
            pl.BlockSpec((1, rows, lanes), lambda i: (i, 0, 0)),
        ],
        out_specs=pl.BlockSpec((1, 8, 128), lambda i: (i, 0, 0)),
        scratch_shapes=[
            pltpu.VMEM((_NBINS, 128), jnp.float32),
            pltpu.VMEM((_NBINS, 128), jnp.float32),
        ],
        compiler_params=pltpu.CompilerParams(
            dimension_semantics=("parallel",),
        ),
        name="kl_soft_hist",
    )(p3, t3)


def kernel(pred, target):
    B = pred.shape[0]
    n = pred.size // B
    lanes = 128
    rows = n // lanes
    p3 = pred.reshape(B, rows, lanes)
    t3 = target.reshape(B, rows, lanes)

    out = _kl_pallas(p3, t3)

    return _W * jnp.mean(out[:, 0, 0])
